# baseline (speedup 1.0000x reference)
import functools

import jax
import jax.numpy as jnp
from jax import lax
from jax.experimental import pallas as pl
from jax.experimental.pallas import tpu as pltpu
from jax.experimental.pallas import tpu_sc as plsc

NV = 6890
NVP = 6912
THR = 0.3
B = 32
NC = 2
NS = 16
LANES = 16
NPIX = 4 * 256 * 256
PIX_PER_TILE = NPIX // NS
C_PX = 512
C_PC = 3 * C_PX
N_CHUNKS = PIX_PER_TILE // C_PX
GRP = 16
NVS = 6913
ACC_N = 110720


_GATHER_DNUMS = lax.GatherDimensionNumbers(
    offset_dims=(), collapsed_slice_dims=(0,), start_index_map=(0,))


def _take(vec, lane):
    idx = jnp.full((LANES, 1), lane, jnp.int32)
    return lax.gather(vec, idx, _GATHER_DNUMS, (1,),
                      mode=lax.GatherScatterMode.PROMISE_IN_BOUNDS)


def _sc_body(msk_hbm, vtx_hbm, bary_hbm, part_hbm,
             msk_b0, vtx_b0, bc_b0, sem0, msk_b1, vtx_b1, bc_b1, sem1,
             acc, stage):
    c = lax.axis_index("c")
    s = lax.axis_index("s")
    iota = lax.iota(jnp.int32, LANES)
    iota_acc = iota * NVS
    lanebit = jnp.int32(1) << (c * LANES + iota)
    zeros = jnp.zeros((LANES,), jnp.float32)
    zero = jnp.int32(0)

    def zero_cols(j, carry):
        for r in range(8):
            acc[pl.ds((j * 8 + r) * LANES, LANES)] = zeros
        return carry

    lax.fori_loop(0, ACC_N // (8 * LANES), zero_cols, 0)

    pc0 = s * (3 * PIX_PER_TILE)
    px0 = s * PIX_PER_TILE
    bufs = ((msk_b0, vtx_b0, bc_b0, sem0), (msk_b1, vtx_b1, bc_b1, sem1))

    def issue(k, bf):
        msk_b, vtx_b, bc_b, sem = bf
        pltpu.async_copy(msk_hbm.at[pl.ds(px0 + k * C_PX, C_PX)], msk_b, sem)
        pltpu.async_copy(vtx_hbm.at[pl.ds(pc0 + k * C_PC, C_PC)], vtx_b, sem)
        pltpu.async_copy(bary_hbm.at[pl.ds(pc0 + k * C_PC, C_PC)], bc_b, sem)

    def drain(bf):
        msk_b, vtx_b, bc_b, sem = bf
        pltpu.make_async_copy(msk_hbm.at[pl.ds(0, C_PX)], msk_b, sem).wait()
        pltpu.make_async_copy(vtx_hbm.at[pl.ds(0, C_PC)], vtx_b, sem).wait()
        pltpu.make_async_copy(bary_hbm.at[pl.ds(0, C_PC)], bc_b, sem).wait()

    issue(0, bufs[0])
    issue(1, bufs[1])

    def chunk(k, carry):
        for t in range(2):
            kk = k + t
            msk_b, vtx_b, bc_b, sem = bufs[t]
            drain(bufs[t])

            @plsc.parallel_loop(0, C_PX // GRP, unroll=2)
            def grp(g):
                p0 = g * GRP
                vt = [vtx_b[pl.ds(p0 * 3 + 16 * u, 16)] for u in range(3)]
                bc = [bc_b[pl.ds(p0 * 3 + 16 * u, 16)] for u in range(3)]
                for i in range(GRP):
                    w = plsc.load_gather(
                        msk_b, [jnp.full((LANES,), p0 + i, jnp.int32)])
                    msk = (w & lanebit) != zero
                    for j in range(3):
                        q = 3 * i + j
                        v_s = _take(vt[q // 16], q % 16)
                        if j == 2:
                            b_s = _take(bc[q // 16], q % 16)
                        else:
                            b_s = plsc.load_gather(
                                bc_b,
                                [jnp.full((LANES,), p0 * 3 + q, jnp.int32)])
                        plsc.addupdate_scatter(
                            acc, [iota_acc + v_s], b_s, mask=msk)

            @pl.when(kk + 2 < N_CHUNKS)
            def _():
                issue(kk + 2, bufs[t])
        return carry

    lax.fori_loop(0, N_CHUNKS // 2, lambda k, cr: chunk(k * 2, cr), 0)

    for l in range(LANES):
        @pl.loop(0, NVP // LANES)
        def destride(kq):
            stage[pl.ds(kq * LANES, LANES)] = acc[pl.ds(l * NVS + kq * LANES, LANES)]

        pltpu.sync_copy(stage, part_hbm.at[s, c * LANES + l])


_sc_scatter = functools.partial(
    pl.kernel,
    out_type=jax.ShapeDtypeStruct((NS, B, NVP), jnp.float32),
    mesh=plsc.VectorSubcoreMesh(
        core_axis_name="c", subcore_axis_name="s", num_cores=NC, num_subcores=NS
    ),
    scratch_types=[
        pltpu.VMEM((C_PX,), jnp.int32),
        pltpu.VMEM((C_PC,), jnp.int32),
        pltpu.VMEM((C_PC,), jnp.float32),
        pltpu.SemaphoreType.DMA,
        pltpu.VMEM((C_PX,), jnp.int32),
        pltpu.VMEM((C_PC,), jnp.int32),
        pltpu.VMEM((C_PC,), jnp.float32),
        pltpu.SemaphoreType.DMA,
        pltpu.VMEM((ACC_N,), jnp.float32),
        pltpu.VMEM((NVP,), jnp.float32),
    ],
    compiler_params=pltpu.CompilerParams(
        needs_layout_passes=False,
        disable_bounds_checks=True,
    ),
)(_sc_body)


def _tc_body(part_ref, out_ref):
    ssum = jnp.sum(part_ref[...], axis=0)
    out_ref[...] = (ssum > 0.0).astype(jnp.float32)


_TC_BLK = 1152


def _tc_reduce(part):
    return pl.pallas_call(
        _tc_body,
        grid=(NVP // _TC_BLK,),
        in_specs=[pl.BlockSpec((NS, B, _TC_BLK), lambda i: (0, 0, i))],
        out_specs=pl.BlockSpec((B, _TC_BLK), lambda i: (0, i)),
        out_shape=jax.ShapeDtypeStruct((B, NVP), jnp.float32),
    )(part)


def kernel(seg_maps, pixel_to_vertex_map, bary_coord_map):
    bits = jnp.left_shift(
        (seg_maps > THR).astype(jnp.int32),
        jnp.arange(B, dtype=jnp.int32).reshape(B, 1, 1, 1),
    )
    mask_words = jnp.sum(bits, axis=0, dtype=jnp.int32).reshape(-1)
    vtx = pixel_to_vertex_map.reshape(-1)
    bc = bary_coord_map.reshape(-1) - THR
    part = _sc_scatter(mask_words, vtx, bc)
    out = _tc_reduce(part)
    return out[:, :NV]


# speedup vs baseline: 1.0811x; 1.0811x over previous
import functools

import jax
import jax.numpy as jnp
from jax import lax
from jax.experimental import pallas as pl
from jax.experimental.pallas import tpu as pltpu
from jax.experimental.pallas import tpu_sc as plsc

NV = 6890
NVP = 6912
THR = 0.3
B = 32
NC = 2
NS = 16
LANES = 16
NPIX = 4 * 256 * 256
PIX_PER_TILE = NPIX // NS
C_PX = 512
C_PC = 3 * C_PX
N_CHUNKS = PIX_PER_TILE // C_PX
GRP = 16
NVS = 6913
ACC_N = 110720


_GATHER_DNUMS = lax.GatherDimensionNumbers(
    offset_dims=(), collapsed_slice_dims=(0,), start_index_map=(0,))


def _take(vec, lane):
    idx = jnp.full((LANES, 1), lane, jnp.int32)
    return lax.gather(vec, idx, _GATHER_DNUMS, (1,),
                      mode=lax.GatherScatterMode.PROMISE_IN_BOUNDS)


def _sc_body(msk_hbm, vtx_hbm, bary_hbm, part_hbm,
             msk_b0, vtx_b0, bc_b0, sem0, msk_b1, vtx_b1, bc_b1, sem1,
             acc, stage):
    c = lax.axis_index("c")
    s = lax.axis_index("s")
    iota = lax.iota(jnp.int32, LANES)
    iota_acc = iota * NVS
    lanebit = jnp.int32(1) << (c * LANES + iota)
    zeros = jnp.zeros((LANES,), jnp.float32)
    zero = jnp.int32(0)

    def zero_cols(j, carry):
        for r in range(8):
            acc[pl.ds((j * 8 + r) * LANES, LANES)] = zeros
        return carry

    lax.fori_loop(0, ACC_N // (8 * LANES), zero_cols, 0)

    pc0 = s * (3 * PIX_PER_TILE)
    px0 = s * PIX_PER_TILE
    bufs = ((msk_b0, vtx_b0, bc_b0, sem0), (msk_b1, vtx_b1, bc_b1, sem1))

    def issue(k, bf):
        msk_b, vtx_b, bc_b, sem = bf
        pltpu.async_copy(msk_hbm.at[pl.ds(px0 + k * C_PX, C_PX)], msk_b, sem)
        pltpu.async_copy(vtx_hbm.at[pl.ds(pc0 + k * C_PC, C_PC)], vtx_b, sem)
        pltpu.async_copy(bary_hbm.at[pl.ds(pc0 + k * C_PC, C_PC)], bc_b, sem)

    def drain(bf):
        msk_b, vtx_b, bc_b, sem = bf
        pltpu.make_async_copy(msk_hbm.at[pl.ds(0, C_PX)], msk_b, sem).wait()
        pltpu.make_async_copy(vtx_hbm.at[pl.ds(0, C_PC)], vtx_b, sem).wait()
        pltpu.make_async_copy(bary_hbm.at[pl.ds(0, C_PC)], bc_b, sem).wait()

    issue(0, bufs[0])
    issue(1, bufs[1])

    def chunk(k, carry):
        for t in range(2):
            kk = k + t
            msk_b, vtx_b, bc_b, sem = bufs[t]
            drain(bufs[t])

            @plsc.parallel_loop(0, C_PX // GRP, unroll=2)
            def grp(g):
                p0 = g * GRP
                words = msk_b[pl.ds(p0, LANES)]
                vt = [vtx_b[pl.ds(p0 * 3 + 16 * u, 16)] for u in range(3)]
                bc = [bc_b[pl.ds(p0 * 3 + 16 * u, 16)] for u in range(3)]
                for i in range(GRP):
                    w = _take(words, i)
                    msk = (w & lanebit) != zero
                    for j in range(3):
                        q = 3 * i + j
                        v_s = _take(vt[q // 16], q % 16)
                        b_s = _take(bc[q // 16], q % 16)
                        plsc.addupdate_scatter(
                            acc, [iota_acc + v_s], b_s, mask=msk)

            @pl.when(kk + 2 < N_CHUNKS)
            def _():
                issue(kk + 2, bufs[t])
        return carry

    lax.fori_loop(0, N_CHUNKS // 2, lambda k, cr: chunk(k * 2, cr), 0)

    for l in range(LANES):
        @pl.loop(0, NVP // LANES)
        def destride(kq):
            stage[pl.ds(kq * LANES, LANES)] = acc[pl.ds(l * NVS + kq * LANES, LANES)]

        pltpu.sync_copy(stage, part_hbm.at[s, c * LANES + l])


_sc_scatter = functools.partial(
    pl.kernel,
    out_type=jax.ShapeDtypeStruct((NS, B, NVP), jnp.float32),
    mesh=plsc.VectorSubcoreMesh(
        core_axis_name="c", subcore_axis_name="s", num_cores=NC, num_subcores=NS
    ),
    scratch_types=[
        pltpu.VMEM((C_PX,), jnp.int32),
        pltpu.VMEM((C_PC,), jnp.int32),
        pltpu.VMEM((C_PC,), jnp.float32),
        pltpu.SemaphoreType.DMA,
        pltpu.VMEM((C_PX,), jnp.int32),
        pltpu.VMEM((C_PC,), jnp.int32),
        pltpu.VMEM((C_PC,), jnp.float32),
        pltpu.SemaphoreType.DMA,
        pltpu.VMEM((ACC_N,), jnp.float32),
        pltpu.VMEM((NVP,), jnp.float32),
    ],
    compiler_params=pltpu.CompilerParams(
        needs_layout_passes=False,
        disable_bounds_checks=True,
    ),
)(_sc_body)


def _tc_body(part_ref, out_ref):
    ssum = jnp.sum(part_ref[...], axis=0)
    out_ref[...] = (ssum > 0.0).astype(jnp.float32)


_TC_BLK = 1152


def _tc_reduce(part):
    return pl.pallas_call(
        _tc_body,
        grid=(NVP // _TC_BLK,),
        in_specs=[pl.BlockSpec((NS, B, _TC_BLK), lambda i: (0, 0, i))],
        out_specs=pl.BlockSpec((B, _TC_BLK), lambda i: (0, i)),
        out_shape=jax.ShapeDtypeStruct((B, NVP), jnp.float32),
    )(part)


def kernel(seg_maps, pixel_to_vertex_map, bary_coord_map):
    bits = jnp.left_shift(
        (seg_maps > THR).astype(jnp.int32),
        jnp.arange(B, dtype=jnp.int32).reshape(B, 1, 1, 1),
    )
    mask_words = jnp.sum(bits, axis=0, dtype=jnp.int32).reshape(-1)
    vtx = pixel_to_vertex_map.reshape(-1)
    bc = bary_coord_map.reshape(-1) - THR
    part = _sc_scatter(mask_words, vtx, bc)
    out = _tc_reduce(part)
    return out[:, :NV]
